# baseline (speedup 1.0000x reference)
import functools

import jax
import jax.numpy as jnp
from jax import lax
from jax.experimental import pallas as pl
from jax.experimental.pallas import tpu as pltpu
from jax.experimental.pallas import tpu_sc as plsc

N = 10000
NP = 10240
E = 320000
D = 128
DH = 64
NSUB = 16
NT = NP // NSUB
ET = E // NSUB
CH = 80
NCHUNK = ET // CH
NBUF = 6
BR = 80
NBLK = NT // BR
ZR = 40
NLAYERS = 3
ALPHA = 1.0 / (NLAYERS + 1)


def _rsqrt16(d):
    y = jnp.full((16,), 1.0, jnp.float32)
    for k in range(1, 20):
        y = jnp.where(d >= float(2 ** k), float(2.0 ** (-k / 2.0)), y)
    for _ in range(4):
        y = y * (1.5 - (0.5 * d) * y * y)
    return y


def _sc_body(x_ref, src_ref, dst_ref, out_ref,
             acc_sh, deg_sh, g_hbm,
             src_v, dst_v, ones_v, zbuf, abuf, obuf, dinv_v, degbuf,
             *bufs_and_sems):
    c = lax.axis_index("c")
    s = lax.axis_index("s")
    rowb = bufs_and_sems[:NBUF]
    gsem = bufs_and_sems[NBUF:2 * NBUF]
    ssem = bufs_and_sems[2 * NBUF:3 * NBUF]

    for k in range(CH // 16):
        ones_v[pl.ds(k * 16, 16)] = jnp.full((16,), 1.0, jnp.float32)

    def _zbuf_fill(i, carry):
        for k in range(DH // 16):
            zbuf[i, pl.ds(k * 16, 16)] = jnp.zeros((16,), jnp.float32)
        return carry
    lax.fori_loop(0, ZR, _zbuf_fill, 0)

    def _degbuf_zero(j, carry):
        degbuf[pl.ds(j * 16, 16)] = jnp.zeros((16,), jnp.float32)
        return carry
    lax.fori_loop(0, NT // 16, _degbuf_zero, 0)

    pltpu.sync_copy(src_ref.at[s], src_v)
    pltpu.sync_copy(dst_ref.at[s], dst_v)

    rb = jnp.minimum(N - s * NT, NT) // BR
    nrow = rb * BR
    pltpu.sync_copy(degbuf, deg_sh.at[pl.ds(s * NT, NT)])
    plsc.subcore_barrier()

    @pl.when(s < NSUB - 1)
    def _():
        pltpu.async_copy(
            x_ref.at[pl.ds(s * NT, NT), pl.ds(c * DH, DH)],
            out_ref.at[pl.ds(s * NT, NT), pl.ds(c * DH, DH)], gsem[0])

    @pl.when(s == NSUB - 1)
    def _():
        pltpu.async_copy(
            x_ref.at[pl.ds(s * NT, N - (NSUB - 1) * NT), pl.ds(c * DH, DH)],
            out_ref.at[pl.ds(s * NT, N - (NSUB - 1) * NT),
                       pl.ds(c * DH, DH)], gsem[0])

    def _zero_acc(b, carry):
        @pl.when(b * ZR < nrow)
        def _():
            pltpu.async_copy(
                zbuf, acc_sh.at[pl.ds(s * NT + b * ZR, ZR)], gsem[1])
        return carry
    lax.fori_loop(0, NT // ZR, _zero_acc, 0)

    def _deg_body(jj, carry):
        for p in range(NBUF):
            j = jj * NBUF + p
            pltpu.async_copy(ones_v, deg_sh.at[dst_v.at[j]], ssem[p],
                             add=True)
        for p in range(NBUF):
            j = jj * NBUF + p
            pltpu.make_async_copy(
                ones_v, deg_sh.at[dst_v.at[j]], ssem[p]).wait()
        return carry
    lax.fori_loop(0, NCHUNK // NBUF, _deg_body, 0)
    for p in range(NCHUNK % NBUF):
        j = (NCHUNK // NBUF) * NBUF + p
        pltpu.sync_copy(ones_v, deg_sh.at[dst_v.at[j]], add=True)

    @pl.when(s < NSUB - 1)
    def _():
        pltpu.make_async_copy(
            x_ref.at[pl.ds(s * NT, NT), pl.ds(c * DH, DH)],
            out_ref.at[pl.ds(s * NT, NT), pl.ds(c * DH, DH)],
            gsem[0]).wait()

    @pl.when(s == NSUB - 1)
    def _():
        pltpu.make_async_copy(
            x_ref.at[pl.ds(s * NT, N - (NSUB - 1) * NT), pl.ds(c * DH, DH)],
            out_ref.at[pl.ds(s * NT, N - (NSUB - 1) * NT),
                       pl.ds(c * DH, DH)], gsem[0]).wait()

    def _zero_acc_wait(b, carry):
        @pl.when(b * ZR < nrow)
        def _():
            pltpu.make_async_copy(
                zbuf, acc_sh.at[pl.ds(s * NT + b * ZR, ZR)], gsem[1]).wait()
        return carry
    lax.fori_loop(0, NT // ZR, _zero_acc_wait, 0)
    plsc.subcore_barrier()

    pltpu.sync_copy(deg_sh.at[pl.ds(s * NT, NT)], degbuf)

    def _dinv_body(j, carry):
        d = degbuf[pl.ds(j * 16, 16)]
        y = _rsqrt16(d)
        dinv_v[pl.ds(j * 16, 16)] = jnp.where(d > 0.5, y, 0.0)
        return carry
    lax.fori_loop(0, NT // 16, _dinv_body, 0)

    def _wait_init_writes(base):
        pltpu.make_async_copy(
            obuf, g_hbm.at[c, pl.ds(base, BR)], ssem[1]).wait()

    def _init_block(b, carry):
        base = s * NT + b * BR

        @pl.when(jnp.logical_and(b > 0, base < N))
        def _():
            _wait_init_writes(base - BR)

        @pl.when(base < N)
        def _():
            pltpu.sync_copy(
                x_ref.at[pl.ds(base, BR), pl.ds(c * DH, DH)], abuf)

            def _scale_init(g, carry2):
                dvec = dinv_v[pl.ds(b * BR + g * 16, 16)]
                for i in range(16):
                    dv = dvec[i]
                    r = g * 16 + i
                    for k in range(DH // 16):
                        sl = pl.ds(k * 16, 16)
                        obuf[r, sl] = abuf[r, sl] * dv
                return carry2
            lax.fori_loop(0, BR // 16, _scale_init, 0)

            pltpu.async_copy(obuf, g_hbm.at[c, pl.ds(base, BR)], ssem[1])
        return carry
    lax.fori_loop(0, NBLK, _init_block, 0)
    _wait_init_writes(s * NT + (rb - 1) * BR)
    plsc.subcore_barrier()

    for ell in range(NLAYERS):
        last = ell == NLAYERS - 1

        for p in range(NBUF):
            pltpu.async_copy(g_hbm.at[c].at[src_v.at[p]], rowb[p], gsem[p])

        def _edge_body(jj, carry):
            for p in range(NBUF):
                j = jj * NBUF + p
                pltpu.make_async_copy(
                    g_hbm.at[c].at[src_v.at[j]], rowb[p], gsem[p]).wait()
                pltpu.async_copy(rowb[p], acc_sh.at[dst_v.at[j]], ssem[p],
                                 add=True)
            for p in range(NBUF):
                j = jj * NBUF + p
                jn = lax.rem(j + NBUF, NCHUNK)
                pltpu.make_async_copy(
                    rowb[p], acc_sh.at[dst_v.at[j]], ssem[p]).wait()
                pltpu.async_copy(
                    g_hbm.at[c].at[src_v.at[jn]], rowb[p], gsem[p])
            return carry
        lax.fori_loop(0, NCHUNK // NBUF, _edge_body, 0)
        for p in range(NCHUNK % NBUF):
            j = (NCHUNK // NBUF) * NBUF + p
            pltpu.make_async_copy(
                g_hbm.at[c].at[src_v.at[j]], rowb[p], gsem[p]).wait()
            pltpu.sync_copy(rowb[p], acc_sh.at[dst_v.at[j]], add=True)
        for p in range(NCHUNK % NBUF, NBUF):
            pltpu.make_async_copy(
                g_hbm.at[c].at[src_v.at[p]], rowb[p], gsem[p]).wait()
        plsc.subcore_barrier()

        if not last:
            def _wait_mid_writes(base):
                pltpu.make_async_copy(
                    abuf, g_hbm.at[c, pl.ds(base, BR)], ssem[0]).wait()
                pltpu.make_async_copy(
                    obuf, out_ref.at[pl.ds(base, BR), pl.ds(c * DH, DH)],
                    ssem[1]).wait()
                pltpu.make_async_copy(
                    zbuf, acc_sh.at[pl.ds(base, ZR)], ssem[2]).wait()
                pltpu.make_async_copy(
                    zbuf, acc_sh.at[pl.ds(base + ZR, ZR)], ssem[3]).wait()

            def _mid_block(b, carry):
                base = s * NT + b * BR

                @pl.when(jnp.logical_and(b > 0, base < N))
                def _():
                    _wait_mid_writes(base - BR)

                @pl.when(base < N)
                def _():
                    pltpu.sync_copy(acc_sh.at[pl.ds(base, BR)], abuf)
                    pltpu.sync_copy(
                        out_ref.at[pl.ds(base, BR), pl.ds(c * DH, DH)], obuf)

                    def _scale_mid(g, carry2):
                        dvec = dinv_v[pl.ds(b * BR + g * 16, 16)]
                        for i in range(16):
                            dv = dvec[i]
                            r = g * 16 + i
                            for k in range(DH // 16):
                                sl = pl.ds(k * 16, 16)
                                h = abuf[r, sl] * dv
                                abuf[r, sl] = h * dv
                                obuf[r, sl] = obuf[r, sl] + h
                        return carry2
                    lax.fori_loop(0, BR // 16, _scale_mid, 0)

                    pltpu.async_copy(
                        abuf, g_hbm.at[c, pl.ds(base, BR)], ssem[0])
                    pltpu.async_copy(
                        obuf, out_ref.at[pl.ds(base, BR), pl.ds(c * DH, DH)],
                        ssem[1])
                    pltpu.async_copy(zbuf, acc_sh.at[pl.ds(base, ZR)],
                                     ssem[2])
                    pltpu.async_copy(zbuf, acc_sh.at[pl.ds(base + ZR, ZR)],
                                     ssem[3])
                return carry
            lax.fori_loop(0, NBLK, _mid_block, 0)
            _wait_mid_writes(s * NT + (rb - 1) * BR)
            plsc.subcore_barrier()
        else:
            def _wait_last_writes(base):
                pltpu.make_async_copy(
                    obuf, out_ref.at[pl.ds(base, BR), pl.ds(c * DH, DH)],
                    ssem[1]).wait()

            def _last_block(b, carry):
                base = s * NT + b * BR

                @pl.when(jnp.logical_and(b > 0, base < N))
                def _():
                    _wait_last_writes(base - BR)

                @pl.when(base < N)
                def _():
                    pltpu.sync_copy(acc_sh.at[pl.ds(base, BR)], abuf)
                    pltpu.sync_copy(
                        out_ref.at[pl.ds(base, BR), pl.ds(c * DH, DH)], obuf)

                    def _scale_last(g, carry2):
                        dvec = dinv_v[pl.ds(b * BR + g * 16, 16)]
                        for i in range(16):
                            dv = dvec[i]
                            r = g * 16 + i
                            for k in range(DH // 16):
                                sl = pl.ds(k * 16, 16)
                                h = abuf[r, sl] * dv
                                obuf[r, sl] = (obuf[r, sl] + h) * ALPHA
                        return carry2
                    lax.fori_loop(0, BR // 16, _scale_last, 0)

                    pltpu.async_copy(
                        obuf, out_ref.at[pl.ds(base, BR), pl.ds(c * DH, DH)],
                        ssem[1])
                return carry
            lax.fori_loop(0, NBLK, _last_block, 0)
            _wait_last_writes(s * NT + (rb - 1) * BR)


_sc_kernel = functools.partial(
    pl.kernel,
    out_type=jax.ShapeDtypeStruct((N, D), jnp.float32),
    mesh=plsc.VectorSubcoreMesh(core_axis_name="c", subcore_axis_name="s"),
    compiler_params=pltpu.CompilerParams(use_tc_tiling_on_sc=False),
    scratch_types=[
        pltpu.VMEM_SHARED((NP, DH), jnp.float32),
        pltpu.VMEM_SHARED((NP,), jnp.float32),
        pltpu.HBM((2, NP, DH), jnp.float32),
        pltpu.VMEM((NCHUNK, CH), jnp.int32),
        pltpu.VMEM((NCHUNK, CH), jnp.int32),
        pltpu.VMEM((CH,), jnp.float32),
        pltpu.VMEM((ZR, DH), jnp.float32),
        pltpu.VMEM((BR, DH), jnp.float32),
        pltpu.VMEM((BR, DH), jnp.float32),
        pltpu.VMEM((NT,), jnp.float32),
        pltpu.VMEM((NT,), jnp.float32),
    ] + [pltpu.VMEM((CH, DH), jnp.float32)] * NBUF
      + [pltpu.SemaphoreType.DMA] * (2 * NBUF),
)(_sc_body)


@jax.jit
def kernel(x, edge_index):
    src_r = edge_index[0].reshape(NSUB, NCHUNK, CH)
    dst_r = edge_index[1].reshape(NSUB, NCHUNK, CH)
    return _sc_kernel(x, src_r, dst_r)


# speedup vs baseline: 1.3782x; 1.3782x over previous
import functools

import jax
import jax.numpy as jnp
from jax import lax
from jax.experimental import pallas as pl
from jax.experimental.pallas import tpu as pltpu
from jax.experimental.pallas import tpu_sc as plsc

N = 10000
NP = 10240
E = 320000
D = 128
DH = 64
NSUB = 16
NT = NP // NSUB
ET = E // NSUB
CH = 80
NCHUNK = ET // CH
NBUF = 6
BR = 80
NBLK = NT // BR
ZR = 40
NLAYERS = 3
ALPHA = 1.0 / (NLAYERS + 1)


def _rsqrt16(d):
    y = jnp.full((16,), 1.0, jnp.float32)
    for k in range(1, 20):
        y = jnp.where(d >= float(2 ** k), float(2.0 ** (-k / 2.0)), y)
    for _ in range(4):
        y = y * (1.5 - (0.5 * d) * y * y)
    return y


def _sc_body(x_ref, src_ref, dst_ref, out_ref,
             acc_sh, deg_sh, g_hbm,
             src_v, dst_v, ones_v, zbuf, abuf, obuf, dinv_v, degbuf,
             *bufs_and_sems):
    c = lax.axis_index("c")
    s = lax.axis_index("s")
    rowb = bufs_and_sems[:NBUF]
    gsem = bufs_and_sems[NBUF:2 * NBUF]
    ssem = bufs_and_sems[2 * NBUF:3 * NBUF]

    for k in range(CH // 16):
        ones_v[pl.ds(k * 16, 16)] = jnp.full((16,), 1.0, jnp.float32)

    def _zbuf_fill(i, carry):
        for k in range(DH // 16):
            zbuf[i, pl.ds(k * 16, 16)] = jnp.zeros((16,), jnp.float32)
        return carry
    lax.fori_loop(0, ZR, _zbuf_fill, 0)

    def _degbuf_zero(j, carry):
        degbuf[pl.ds(j * 16, 16)] = jnp.zeros((16,), jnp.float32)
        return carry
    lax.fori_loop(0, NT // 16, _degbuf_zero, 0)

    pltpu.sync_copy(src_ref.at[s], src_v)
    pltpu.sync_copy(dst_ref.at[s], dst_v)

    pltpu.sync_copy(degbuf, deg_sh.at[pl.ds(s * NT, NT)])
    plsc.subcore_barrier()

    def _deg_body(jj, carry):
        for p in range(NBUF):
            j = jj * NBUF + p
            pltpu.async_copy(ones_v, deg_sh.at[dst_v.at[j]], ssem[p],
                             add=True)
        for p in range(NBUF):
            j = jj * NBUF + p
            pltpu.make_async_copy(
                ones_v, deg_sh.at[dst_v.at[j]], ssem[p]).wait()
        return carry
    lax.fori_loop(0, NCHUNK // NBUF, _deg_body, 0)
    for p in range(NCHUNK % NBUF):
        j = (NCHUNK // NBUF) * NBUF + p
        pltpu.sync_copy(ones_v, deg_sh.at[dst_v.at[j]], add=True)
    plsc.subcore_barrier()

    pltpu.sync_copy(deg_sh.at[pl.ds(s * NT, NT)], degbuf)

    def _dinv_body(j, carry):
        d = degbuf[pl.ds(j * 16, 16)]
        y = _rsqrt16(d)
        dinv_v[pl.ds(j * 16, 16)] = jnp.where(d > 0.5, y, 0.0)
        return carry
    lax.fori_loop(0, NT // 16, _dinv_body, 0)

    rb = jnp.minimum(N - s * NT, NT) // BR

    def _wait_init_writes(base):
        pltpu.make_async_copy(
            abuf, out_ref.at[pl.ds(base, BR), pl.ds(c * DH, DH)],
            ssem[0]).wait()
        pltpu.make_async_copy(
            obuf, g_hbm.at[c, pl.ds(base, BR)], ssem[1]).wait()
        pltpu.make_async_copy(
            zbuf, acc_sh.at[pl.ds(base, ZR)], ssem[2]).wait()
        pltpu.make_async_copy(
            zbuf, acc_sh.at[pl.ds(base + ZR, ZR)], ssem[3]).wait()

    def _init_block(b, carry):
        base = s * NT + b * BR

        @pl.when(jnp.logical_and(b > 0, base < N))
        def _():
            _wait_init_writes(base - BR)

        @pl.when(base < N)
        def _():
            pltpu.sync_copy(
                x_ref.at[pl.ds(base, BR), pl.ds(c * DH, DH)], abuf)
            pltpu.async_copy(
                abuf, out_ref.at[pl.ds(base, BR), pl.ds(c * DH, DH)],
                ssem[0])

            def _scale_init(g, carry2):
                dvec = dinv_v[pl.ds(b * BR + g * 16, 16)]
                for i in range(16):
                    dv = dvec[i]
                    r = g * 16 + i
                    for k in range(DH // 16):
                        sl = pl.ds(k * 16, 16)
                        obuf[r, sl] = abuf[r, sl] * dv
                return carry2
            lax.fori_loop(0, BR // 16, _scale_init, 0)

            pltpu.async_copy(obuf, g_hbm.at[c, pl.ds(base, BR)], ssem[1])
            pltpu.async_copy(zbuf, acc_sh.at[pl.ds(base, ZR)], ssem[2])
            pltpu.async_copy(zbuf, acc_sh.at[pl.ds(base + ZR, ZR)], ssem[3])
        return carry
    lax.fori_loop(0, NBLK, _init_block, 0)
    _wait_init_writes(s * NT + (rb - 1) * BR)
    plsc.subcore_barrier()

    for ell in range(NLAYERS):
        last = ell == NLAYERS - 1

        for p in range(NBUF):
            pltpu.async_copy(g_hbm.at[c].at[src_v.at[p]], rowb[p], gsem[p])

        def _edge_body(jj, carry):
            for p in range(NBUF):
                j = jj * NBUF + p
                pltpu.make_async_copy(
                    g_hbm.at[c].at[src_v.at[j]], rowb[p], gsem[p]).wait()
                pltpu.async_copy(rowb[p], acc_sh.at[dst_v.at[j]], ssem[p],
                                 add=True)
            for p in range(NBUF):
                j = jj * NBUF + p
                jn = lax.rem(j + NBUF, NCHUNK)
                pltpu.make_async_copy(
                    rowb[p], acc_sh.at[dst_v.at[j]], ssem[p]).wait()
                pltpu.async_copy(
                    g_hbm.at[c].at[src_v.at[jn]], rowb[p], gsem[p])
            return carry
        lax.fori_loop(0, NCHUNK // NBUF, _edge_body, 0)
        for p in range(NCHUNK % NBUF):
            j = (NCHUNK // NBUF) * NBUF + p
            pltpu.make_async_copy(
                g_hbm.at[c].at[src_v.at[j]], rowb[p], gsem[p]).wait()
            pltpu.sync_copy(rowb[p], acc_sh.at[dst_v.at[j]], add=True)
        for p in range(NCHUNK % NBUF, NBUF):
            pltpu.make_async_copy(
                g_hbm.at[c].at[src_v.at[p]], rowb[p], gsem[p]).wait()
        plsc.subcore_barrier()

        if not last:
            def _wait_mid_writes(base):
                pltpu.make_async_copy(
                    abuf, g_hbm.at[c, pl.ds(base, BR)], ssem[0]).wait()
                pltpu.make_async_copy(
                    obuf, out_ref.at[pl.ds(base, BR), pl.ds(c * DH, DH)],
                    ssem[1]).wait()
                pltpu.make_async_copy(
                    zbuf, acc_sh.at[pl.ds(base, ZR)], ssem[2]).wait()
                pltpu.make_async_copy(
                    zbuf, acc_sh.at[pl.ds(base + ZR, ZR)], ssem[3]).wait()

            def _mid_block(b, carry):
                base = s * NT + b * BR

                @pl.when(jnp.logical_and(b > 0, base < N))
                def _():
                    _wait_mid_writes(base - BR)

                @pl.when(base < N)
                def _():
                    pltpu.sync_copy(acc_sh.at[pl.ds(base, BR)], abuf)
                    pltpu.sync_copy(
                        out_ref.at[pl.ds(base, BR), pl.ds(c * DH, DH)], obuf)

                    def _scale_mid(g, carry2):
                        dvec = dinv_v[pl.ds(b * BR + g * 16, 16)]
                        for i in range(16):
                            dv = dvec[i]
                            r = g * 16 + i
                            for k in range(DH // 16):
                                sl = pl.ds(k * 16, 16)
                                h = abuf[r, sl] * dv
                                abuf[r, sl] = h * dv
                                obuf[r, sl] = obuf[r, sl] + h
                        return carry2
                    lax.fori_loop(0, BR // 16, _scale_mid, 0)

                    pltpu.async_copy(
                        abuf, g_hbm.at[c, pl.ds(base, BR)], ssem[0])
                    pltpu.async_copy(
                        obuf, out_ref.at[pl.ds(base, BR), pl.ds(c * DH, DH)],
                        ssem[1])
                    pltpu.async_copy(zbuf, acc_sh.at[pl.ds(base, ZR)],
                                     ssem[2])
                    pltpu.async_copy(zbuf, acc_sh.at[pl.ds(base + ZR, ZR)],
                                     ssem[3])
                return carry
            lax.fori_loop(0, NBLK, _mid_block, 0)
            _wait_mid_writes(s * NT + (rb - 1) * BR)
            plsc.subcore_barrier()
        else:
            def _wait_last_writes(base):
                pltpu.make_async_copy(
                    obuf, out_ref.at[pl.ds(base, BR), pl.ds(c * DH, DH)],
                    ssem[1]).wait()

            def _last_block(b, carry):
                base = s * NT + b * BR

                @pl.when(jnp.logical_and(b > 0, base < N))
                def _():
                    _wait_last_writes(base - BR)

                @pl.when(base < N)
                def _():
                    pltpu.sync_copy(acc_sh.at[pl.ds(base, BR)], abuf)
                    pltpu.sync_copy(
                        out_ref.at[pl.ds(base, BR), pl.ds(c * DH, DH)], obuf)

                    def _scale_last(g, carry2):
                        dvec = dinv_v[pl.ds(b * BR + g * 16, 16)]
                        for i in range(16):
                            dv = dvec[i]
                            r = g * 16 + i
                            for k in range(DH // 16):
                                sl = pl.ds(k * 16, 16)
                                h = abuf[r, sl] * dv
                                obuf[r, sl] = (obuf[r, sl] + h) * ALPHA
                        return carry2
                    lax.fori_loop(0, BR // 16, _scale_last, 0)

                    pltpu.async_copy(
                        obuf, out_ref.at[pl.ds(base, BR), pl.ds(c * DH, DH)],
                        ssem[1])
                return carry
            lax.fori_loop(0, NBLK, _last_block, 0)
            _wait_last_writes(s * NT + (rb - 1) * BR)


_sc_kernel = functools.partial(
    pl.kernel,
    out_type=jax.ShapeDtypeStruct((N, D), jnp.float32),
    mesh=plsc.VectorSubcoreMesh(core_axis_name="c", subcore_axis_name="s"),
    compiler_params=pltpu.CompilerParams(use_tc_tiling_on_sc=False),
    scratch_types=[
        pltpu.VMEM_SHARED((NP, DH), jnp.float32),
        pltpu.VMEM_SHARED((NP,), jnp.float32),
        pltpu.HBM((2, NP, DH), jnp.float32),
        pltpu.VMEM((NCHUNK, CH), jnp.int32),
        pltpu.VMEM((NCHUNK, CH), jnp.int32),
        pltpu.VMEM((CH,), jnp.float32),
        pltpu.VMEM((ZR, DH), jnp.float32),
        pltpu.VMEM((BR, DH), jnp.float32),
        pltpu.VMEM((BR, DH), jnp.float32),
        pltpu.VMEM((NT,), jnp.float32),
        pltpu.VMEM((NT,), jnp.float32),
    ] + [pltpu.VMEM((CH, DH), jnp.float32)] * NBUF
      + [pltpu.SemaphoreType.DMA] * (2 * NBUF),
)(_sc_body)


@jax.jit
def kernel(x, edge_index):
    src_r = edge_index[0].reshape(NSUB, NCHUNK, CH)
    dst_r = edge_index[1].reshape(NSUB, NCHUNK, CH)
    return _sc_kernel(x, src_r, dst_r)
